# Initial kernel scaffold; baseline (speedup 1.0000x reference)
import jax
import jax.numpy as jnp
from jax.experimental import pallas as pl


def kernel(x, edge_index, W, b):
    raise NotImplementedError("write your pallas kernel here")


# speedup vs baseline: 26.1611x; 26.1611x over previous
import functools

import jax
import jax.numpy as jnp
from jax import lax
from jax.experimental import pallas as pl
from jax.experimental.pallas import tpu as pltpu
from jax.experimental.pallas import tpu_sc as plsc

NC = 2
NS = 16
L = 16
NW = NC * NS

CH = 80
R = 512


def _deg_body(npad, nch, dst_hbm, z_hbm, deg_hbm, dsti, ones_v, deg_sp):
    pers = npad // NS
    c = lax.axis_index("c")
    s = lax.axis_index("s")
    w = c * NS + s

    def ibody(i, carry):
        ones_v[pl.ds(i * L, L)] = jnp.ones((L,), jnp.float32)
        return carry

    lax.fori_loop(0, CH // L, ibody, 0)
    pltpu.sync_copy(z_hbm.at[pl.ds(s * pers, pers)],
                    deg_sp.at[pl.ds(s * pers, pers)])
    pltpu.sync_copy(dst_hbm.at[w], dsti)
    plsc.subcore_barrier()

    def jbody(j, carry):
        pltpu.sync_copy(ones_v, deg_sp.at[dsti.at[j]], add=True)
        return carry

    lax.fori_loop(0, nch, jbody, 0)
    plsc.subcore_barrier()
    pltpu.sync_copy(deg_sp.at[pl.ds(s * pers, pers)],
                    deg_hbm.at[c, pl.ds(s * pers, pers)])


def _scatter_body(npad, nch, d, x2_hbm, src_hbm, dst_hbm, z_hbm, agg_hbm,
                  srci, dsti, rows, sem, agg_sp):
    pers = npad // NS
    c = lax.axis_index("c")
    s = lax.axis_index("s")
    w = c * NS + s

    pltpu.sync_copy(z_hbm.at[pl.ds(s * pers, pers)],
                    agg_sp.at[pl.ds(s * pers, pers)])
    pltpu.sync_copy(src_hbm.at[w], srci)
    pltpu.sync_copy(dst_hbm.at[w], dsti)
    plsc.subcore_barrier()

    def jbody(j, carry):
        pltpu.async_copy(x2_hbm.at[srci.at[j]], rows, sem).wait()
        pltpu.sync_copy(rows, agg_sp.at[dsti.at[j]], add=True)
        return carry

    lax.fori_loop(0, nch, jbody, 0)
    plsc.subcore_barrier()
    pltpu.sync_copy(agg_sp.at[pl.ds(s * pers, pers)],
                    agg_hbm.at[c, pl.ds(s * pers, pers)])


def _scale_body(x_ref, degt_ref, o_ref):
    d = degt_ref[...]
    dinv = lax.rsqrt(d[:, 0:1] + d[:, 1:2] + 1.0)
    o_ref[...] = x_ref[...] * dinv


def _final_body(agg_ref, x2_ref, degt_ref, w_ref, b_ref, o_ref):
    d = degt_ref[...]
    dinv = lax.rsqrt(d[:, 0:1] + d[:, 1:2] + 1.0)
    pre = (agg_ref[0] + agg_ref[1] + x2_ref[...]) * dinv
    o_ref[...] = (jnp.dot(pre, w_ref[...], preferred_element_type=jnp.float32)
                  + b_ref[...])


@jax.jit
def kernel(x, edge_index, W, b):
    n, d = x.shape
    e = edge_index.shape[1]
    assert e % (NW * CH) == 0
    nch = e // (NW * CH)
    npad = ((n + NS * L - 1) // (NS * L)) * (NS * L)

    src3 = edge_index[0].reshape(NW, nch, CH)
    dst3 = edge_index[1].reshape(NW, nch, CH)
    z1 = jnp.zeros((npad,), jnp.float32)
    z2 = jnp.zeros((npad, d), jnp.float32)

    mesh = plsc.VectorSubcoreMesh(core_axis_name="c", subcore_axis_name="s")

    deg = pl.kernel(
        functools.partial(_deg_body, npad, nch),
        out_type=jax.ShapeDtypeStruct((NC, npad), jnp.float32),
        mesh=mesh,
        scratch_types=[
            pltpu.VMEM((nch, CH), jnp.int32),
            pltpu.VMEM((CH,), jnp.float32),
            pltpu.VMEM_SHARED((npad,), jnp.float32),
        ],
    )(dst3, z1)
    degt = deg.T

    nblk = npad // R
    x2 = pl.pallas_call(
        _scale_body,
        grid=(nblk,),
        in_specs=[
            pl.BlockSpec((R, d), lambda i: (i, 0)),
            pl.BlockSpec((R, NC), lambda i: (i, 0)),
        ],
        out_specs=pl.BlockSpec((R, d), lambda i: (i, 0)),
        out_shape=jax.ShapeDtypeStruct((n, d), jnp.float32),
    )(x, degt)

    agg = pl.kernel(
        functools.partial(_scatter_body, npad, nch, d),
        out_type=jax.ShapeDtypeStruct((NC, npad, d), jnp.float32),
        mesh=mesh,
        scratch_types=[
            pltpu.VMEM((nch, CH), jnp.int32),
            pltpu.VMEM((nch, CH), jnp.int32),
            pltpu.VMEM((CH, d), jnp.float32),
            pltpu.SemaphoreType.DMA,
            pltpu.VMEM_SHARED((npad, d), jnp.float32),
        ],
    )(x2, src3, dst3, z2)

    out = pl.pallas_call(
        _final_body,
        grid=(nblk,),
        in_specs=[
            pl.BlockSpec((NC, R, d), lambda i: (0, i, 0)),
            pl.BlockSpec((R, d), lambda i: (i, 0)),
            pl.BlockSpec((R, NC), lambda i: (i, 0)),
            pl.BlockSpec((d, d), lambda i: (0, 0)),
            pl.BlockSpec((1, d), lambda i: (0, 0)),
        ],
        out_specs=pl.BlockSpec((R, d), lambda i: (i, 0)),
        out_shape=jax.ShapeDtypeStruct((n, d), jnp.float32),
    )(agg, x2, degt, W, b.reshape(1, d))

    return out


# speedup vs baseline: 31.3010x; 1.1965x over previous
import functools

import jax
import jax.numpy as jnp
from jax import lax
from jax.experimental import pallas as pl
from jax.experimental.pallas import tpu as pltpu
from jax.experimental.pallas import tpu_sc as plsc

NC = 2
NS = 16
L = 16
NW = NC * NS

CH = 80
R = 512


def _deg_body(npad, nch, dst_hbm, z_hbm, deg_hbm, dsti, ones_v, deg_sp):
    pers = npad // NS
    c = lax.axis_index("c")
    s = lax.axis_index("s")
    w = c * NS + s

    def ibody(i, carry):
        ones_v[pl.ds(i * L, L)] = jnp.ones((L,), jnp.float32)
        return carry

    lax.fori_loop(0, CH // L, ibody, 0)
    pltpu.sync_copy(z_hbm.at[pl.ds(s * pers, pers)],
                    deg_sp.at[pl.ds(s * pers, pers)])
    pltpu.sync_copy(dst_hbm.at[w], dsti)
    plsc.subcore_barrier()

    def jbody(j, carry):
        pltpu.sync_copy(ones_v, deg_sp.at[dsti.at[j]], add=True)
        return carry

    lax.fori_loop(0, nch, jbody, 0)
    plsc.subcore_barrier()
    pltpu.sync_copy(deg_sp.at[pl.ds(s * pers, pers)],
                    deg_hbm.at[c, pl.ds(s * pers, pers)])


def _scatter_body(npad, nch, d, x2_hbm, ei_hbm, z_hbm, agg_hbm,
                  idx, rows, isem, gsem, agg_sp):
    pers = npad // NS
    c = lax.axis_index("c")
    s = lax.axis_index("s")
    w = c * NS + s

    pltpu.sync_copy(z_hbm.at[pl.ds(s * pers, pers)],
                    agg_sp.at[pl.ds(s * pers, pers)])
    plsc.subcore_barrier()

    pltpu.async_copy(ei_hbm.at[w, 0], idx.at[0], isem).wait()
    pltpu.async_copy(x2_hbm.at[idx.at[0, 0]], rows.at[0], gsem)
    pltpu.async_copy(ei_hbm.at[w, 1], idx.at[1], isem)

    def jbody(j, carry):
        b = lax.rem(j, 2)
        nb = 1 - b
        pltpu.make_async_copy(x2_hbm.at[idx.at[b, 0]], rows.at[b],
                              gsem).wait()

        @pl.when(j + 1 < nch)
        def _():
            pltpu.make_async_copy(ei_hbm.at[w, j + 1], idx.at[nb],
                                  isem).wait()
            pltpu.async_copy(x2_hbm.at[idx.at[nb, 0]], rows.at[nb], gsem)

        pltpu.sync_copy(rows.at[b], agg_sp.at[idx.at[b, 1]], add=True)

        @pl.when(j + 2 < nch)
        def _():
            pltpu.async_copy(ei_hbm.at[w, j + 2], idx.at[b], isem)

        return carry

    lax.fori_loop(0, nch, jbody, 0)
    plsc.subcore_barrier()
    pltpu.sync_copy(agg_sp.at[pl.ds(s * pers, pers)],
                    agg_hbm.at[c, pl.ds(s * pers, pers)])


def _scale_body(x_ref, degt_ref, o_ref):
    d = degt_ref[...]
    dinv = lax.rsqrt(d[:, 0:1] + d[:, 1:2] + 1.0)
    o_ref[...] = x_ref[...] * dinv


def _final_body(agg_ref, x2_ref, degt_ref, w_ref, b_ref, o_ref):
    d = degt_ref[...]
    dinv = lax.rsqrt(d[:, 0:1] + d[:, 1:2] + 1.0)
    pre = (agg_ref[0] + agg_ref[1] + x2_ref[...]) * dinv
    o_ref[...] = (jnp.dot(pre, w_ref[...], preferred_element_type=jnp.float32)
                  + b_ref[...])


@jax.jit
def kernel(x, edge_index, W, b):
    n, d = x.shape
    e = edge_index.shape[1]
    assert e % (NW * CH) == 0
    nch = e // (NW * CH)
    npad = ((n + NS * L - 1) // (NS * L)) * (NS * L)

    src3 = edge_index[0].reshape(NW, nch, CH)
    dst3 = edge_index[1].reshape(NW, nch, CH)
    z1 = jnp.zeros((npad,), jnp.float32)
    z2 = jnp.zeros((npad, d), jnp.float32)

    mesh = plsc.VectorSubcoreMesh(core_axis_name="c", subcore_axis_name="s")

    deg = pl.kernel(
        functools.partial(_deg_body, npad, nch),
        out_type=jax.ShapeDtypeStruct((NC, npad), jnp.float32),
        mesh=mesh,
        scratch_types=[
            pltpu.VMEM((nch, CH), jnp.int32),
            pltpu.VMEM((CH,), jnp.float32),
            pltpu.VMEM_SHARED((npad,), jnp.float32),
        ],
    )(dst3, z1)
    degt = deg.T

    nblk = npad // R
    x2 = pl.pallas_call(
        _scale_body,
        grid=(nblk,),
        in_specs=[
            pl.BlockSpec((R, d), lambda i: (i, 0)),
            pl.BlockSpec((R, NC), lambda i: (i, 0)),
        ],
        out_specs=pl.BlockSpec((R, d), lambda i: (i, 0)),
        out_shape=jax.ShapeDtypeStruct((n, d), jnp.float32),
    )(x, degt)

    ei3 = jnp.stack([src3, dst3], axis=2)

    agg = pl.kernel(
        functools.partial(_scatter_body, npad, nch, d),
        out_type=jax.ShapeDtypeStruct((NC, npad, d), jnp.float32),
        mesh=mesh,
        scratch_types=[
            pltpu.VMEM((2, 2, CH), jnp.int32),
            pltpu.VMEM((2, CH, d), jnp.float32),
            pltpu.SemaphoreType.DMA,
            pltpu.SemaphoreType.DMA,
            pltpu.VMEM_SHARED((npad, d), jnp.float32),
        ],
    )(x2, ei3, z2)

    out = pl.pallas_call(
        _final_body,
        grid=(nblk,),
        in_specs=[
            pl.BlockSpec((NC, R, d), lambda i: (0, i, 0)),
            pl.BlockSpec((R, d), lambda i: (i, 0)),
            pl.BlockSpec((R, NC), lambda i: (i, 0)),
            pl.BlockSpec((d, d), lambda i: (0, 0)),
            pl.BlockSpec((1, d), lambda i: (0, 0)),
        ],
        out_specs=pl.BlockSpec((R, d), lambda i: (i, 0)),
        out_shape=jax.ShapeDtypeStruct((n, d), jnp.float32),
    )(agg, x2, degt, W, b.reshape(1, d))

    return out


# speedup vs baseline: 31.3384x; 1.0012x over previous
import functools

import jax
import jax.numpy as jnp
from jax import lax
from jax.experimental import pallas as pl
from jax.experimental.pallas import tpu as pltpu
from jax.experimental.pallas import tpu_sc as plsc

NC = 2
NS = 16
L = 16
NW = NC * NS

CH = 80
R = 512


def _deg_body(npad, nch, dst_hbm, z_hbm, deg_hbm, dsti, ones_v, deg_sp):
    pers = npad // NS
    c = lax.axis_index("c")
    s = lax.axis_index("s")
    w = c * NS + s

    def ibody(i, carry):
        ones_v[pl.ds(i * L, L)] = jnp.ones((L,), jnp.float32)
        return carry

    lax.fori_loop(0, CH // L, ibody, 0)
    pltpu.sync_copy(z_hbm.at[pl.ds(s * pers, pers)],
                    deg_sp.at[pl.ds(s * pers, pers)])
    pltpu.sync_copy(dst_hbm.at[w], dsti)
    plsc.subcore_barrier()

    def jbody(j, carry):
        pltpu.sync_copy(ones_v, deg_sp.at[dsti.at[j]], add=True)
        return carry

    lax.fori_loop(0, nch, jbody, 0)
    plsc.subcore_barrier()
    pltpu.sync_copy(deg_sp.at[pl.ds(s * pers, pers)],
                    deg_hbm.at[c, pl.ds(s * pers, pers)])


def _scatter_body(npad, nch, d, x2_hbm, ei_hbm, z_hbm, agg_hbm,
                  idx, rows, isem, gsem, ssem, agg_sp):
    pers = npad // NS
    c = lax.axis_index("c")
    s = lax.axis_index("s")
    w = c * NS + s

    pltpu.sync_copy(z_hbm.at[pl.ds(s * pers, pers)],
                    agg_sp.at[pl.ds(s * pers, pers)])
    plsc.subcore_barrier()

    pltpu.async_copy(ei_hbm.at[w, 0], idx.at[0], isem).wait()
    pltpu.async_copy(x2_hbm.at[idx.at[0, 0]], rows.at[0], gsem)
    pltpu.async_copy(ei_hbm.at[w, 1], idx.at[1], isem)

    def jbody(j, carry):
        b = lax.rem(j, 2)
        nb = 1 - b
        ib = lax.rem(j, 4)
        pltpu.make_async_copy(x2_hbm.at[idx.at[ib, 0]], rows.at[b],
                              gsem).wait()
        pltpu.async_copy(rows.at[b], agg_sp.at[idx.at[ib, 1]], ssem,
                         add=True)

        @pl.when(j >= 1)
        def _():
            pltpu.make_async_copy(rows.at[nb],
                                  agg_sp.at[idx.at[lax.rem(j + 3, 4), 1]],
                                  ssem).wait()

        @pl.when(j + 1 < nch)
        def _():
            nib = lax.rem(j + 1, 4)
            pltpu.make_async_copy(ei_hbm.at[w, j + 1], idx.at[nib],
                                  isem).wait()
            pltpu.async_copy(x2_hbm.at[idx.at[nib, 0]], rows.at[nb], gsem)

        @pl.when(j + 2 < nch)
        def _():
            pltpu.async_copy(ei_hbm.at[w, j + 2], idx.at[lax.rem(j + 2, 4)],
                             isem)

        return carry

    lax.fori_loop(0, nch, jbody, 0)
    pltpu.make_async_copy(rows.at[lax.rem(nch - 1, 2)],
                          agg_sp.at[idx.at[lax.rem(nch - 1, 4), 1]],
                          ssem).wait()
    plsc.subcore_barrier()
    pltpu.sync_copy(agg_sp.at[pl.ds(s * pers, pers)],
                    agg_hbm.at[c, pl.ds(s * pers, pers)])


def _scale_body(x_ref, degt_ref, o_ref):
    d = degt_ref[...]
    dinv = lax.rsqrt(d[:, 0:1] + d[:, 1:2] + 1.0)
    o_ref[...] = x_ref[...] * dinv


def _final_body(agg_ref, x2_ref, degt_ref, w_ref, b_ref, o_ref):
    d = degt_ref[...]
    dinv = lax.rsqrt(d[:, 0:1] + d[:, 1:2] + 1.0)
    pre = (agg_ref[0] + agg_ref[1] + x2_ref[...]) * dinv
    o_ref[...] = (jnp.dot(pre, w_ref[...], preferred_element_type=jnp.float32)
                  + b_ref[...])


@jax.jit
def kernel(x, edge_index, W, b):
    n, d = x.shape
    e = edge_index.shape[1]
    assert e % (NW * CH) == 0
    nch = e // (NW * CH)
    npad = ((n + NS * L - 1) // (NS * L)) * (NS * L)

    src3 = edge_index[0].reshape(NW, nch, CH)
    dst3 = edge_index[1].reshape(NW, nch, CH)
    z1 = jnp.zeros((npad,), jnp.float32)
    z2 = jnp.zeros((npad, d), jnp.float32)

    mesh = plsc.VectorSubcoreMesh(core_axis_name="c", subcore_axis_name="s")

    deg = pl.kernel(
        functools.partial(_deg_body, npad, nch),
        out_type=jax.ShapeDtypeStruct((NC, npad), jnp.float32),
        mesh=mesh,
        scratch_types=[
            pltpu.VMEM((nch, CH), jnp.int32),
            pltpu.VMEM((CH,), jnp.float32),
            pltpu.VMEM_SHARED((npad,), jnp.float32),
        ],
    )(dst3, z1)
    degt = deg.T

    nblk = npad // R
    x2 = pl.pallas_call(
        _scale_body,
        grid=(nblk,),
        in_specs=[
            pl.BlockSpec((R, d), lambda i: (i, 0)),
            pl.BlockSpec((R, NC), lambda i: (i, 0)),
        ],
        out_specs=pl.BlockSpec((R, d), lambda i: (i, 0)),
        out_shape=jax.ShapeDtypeStruct((n, d), jnp.float32),
    )(x, degt)

    ei3 = jnp.stack([src3, dst3], axis=2)

    agg = pl.kernel(
        functools.partial(_scatter_body, npad, nch, d),
        out_type=jax.ShapeDtypeStruct((NC, npad, d), jnp.float32),
        mesh=mesh,
        scratch_types=[
            pltpu.VMEM((4, 2, CH), jnp.int32),
            pltpu.VMEM((2, CH, d), jnp.float32),
            pltpu.SemaphoreType.DMA,
            pltpu.SemaphoreType.DMA,
            pltpu.SemaphoreType.DMA,
            pltpu.VMEM_SHARED((npad, d), jnp.float32),
        ],
    )(x2, ei3, z2)

    out = pl.pallas_call(
        _final_body,
        grid=(nblk,),
        in_specs=[
            pl.BlockSpec((NC, R, d), lambda i: (0, i, 0)),
            pl.BlockSpec((R, d), lambda i: (i, 0)),
            pl.BlockSpec((R, NC), lambda i: (i, 0)),
            pl.BlockSpec((d, d), lambda i: (0, 0)),
            pl.BlockSpec((1, d), lambda i: (0, 0)),
        ],
        out_specs=pl.BlockSpec((R, d), lambda i: (i, 0)),
        out_shape=jax.ShapeDtypeStruct((n, d), jnp.float32),
    )(agg, x2, degt, W, b.reshape(1, d))

    return out


# speedup vs baseline: 31.3849x; 1.0015x over previous
import functools

import jax
import jax.numpy as jnp
from jax import lax
from jax.experimental import pallas as pl
from jax.experimental.pallas import tpu as pltpu
from jax.experimental.pallas import tpu_sc as plsc

NC = 2
NS = 16
L = 16
NW = NC * NS

CH = 80
R = 512


def _deg_body(npad, nch, dst_hbm, z_hbm, deg_hbm, dsti, ones_v, deg_sp):
    pers = npad // NS
    c = lax.axis_index("c")
    s = lax.axis_index("s")
    w = c * NS + s

    def ibody(i, carry):
        ones_v[pl.ds(i * L, L)] = jnp.ones((L,), jnp.float32)
        return carry

    lax.fori_loop(0, CH // L, ibody, 0)
    pltpu.sync_copy(z_hbm.at[pl.ds(s * pers, pers)],
                    deg_sp.at[pl.ds(s * pers, pers)])
    pltpu.sync_copy(dst_hbm.at[w], dsti)
    plsc.subcore_barrier()

    def jbody(j, carry):
        pltpu.sync_copy(ones_v, deg_sp.at[dsti.at[j]], add=True)
        return carry

    lax.fori_loop(0, nch, jbody, 0)
    plsc.subcore_barrier()
    pltpu.sync_copy(deg_sp.at[pl.ds(s * pers, pers)],
                    deg_hbm.at[c, pl.ds(s * pers, pers)])


def _scatter_body(npad, nch, d, x2_hbm, ei_hbm, z_hbm, agg_hbm,
                  idx, rows, isem, gsem, ssem, agg_sp):
    pers = npad // NS
    c = lax.axis_index("c")
    s = lax.axis_index("s")
    w = c * NS + s

    pltpu.sync_copy(z_hbm.at[pl.ds(s * pers, pers)],
                    agg_sp.at[pl.ds(s * pers, pers)])
    plsc.subcore_barrier()

    pltpu.async_copy(ei_hbm.at[w, 0], idx.at[0], isem).wait()
    pltpu.async_copy(x2_hbm.at[idx.at[0, 0]], rows.at[0], gsem)
    pltpu.async_copy(ei_hbm.at[w, 1], idx.at[1], isem)

    def jbody(j, carry):
        b = lax.rem(j, 2)
        nb = 1 - b
        ib = lax.rem(j, 4)
        pltpu.make_async_copy(x2_hbm.at[idx.at[ib, 0]], rows.at[b],
                              gsem).wait()

        @pl.when(j >= 1)
        def _():
            pltpu.make_async_copy(rows.at[nb],
                                  agg_sp.at[idx.at[lax.rem(j + 3, 4), 1]],
                                  ssem).wait()

        pltpu.async_copy(rows.at[b], agg_sp.at[idx.at[ib, 1]], ssem,
                         add=True)

        @pl.when(j + 1 < nch)
        def _():
            nib = lax.rem(j + 1, 4)
            pltpu.make_async_copy(ei_hbm.at[w, j + 1], idx.at[nib],
                                  isem).wait()
            pltpu.async_copy(x2_hbm.at[idx.at[nib, 0]], rows.at[nb], gsem)

        @pl.when(j + 2 < nch)
        def _():
            pltpu.async_copy(ei_hbm.at[w, j + 2], idx.at[lax.rem(j + 2, 4)],
                             isem)

        return carry

    lax.fori_loop(0, nch, jbody, 0)
    pltpu.make_async_copy(rows.at[lax.rem(nch - 1, 2)],
                          agg_sp.at[idx.at[lax.rem(nch - 1, 4), 1]],
                          ssem).wait()
    plsc.subcore_barrier()
    pltpu.sync_copy(agg_sp.at[pl.ds(s * pers, pers)],
                    agg_hbm.at[c, pl.ds(s * pers, pers)])


def _scale_body(x_ref, degt_ref, o_ref):
    d = degt_ref[...]
    dinv = lax.rsqrt(d[:, 0:1] + d[:, 1:2] + 1.0)
    o_ref[...] = x_ref[...] * dinv


def _final_body(agg_ref, x2_ref, degt_ref, w_ref, b_ref, o_ref):
    d = degt_ref[...]
    dinv = lax.rsqrt(d[:, 0:1] + d[:, 1:2] + 1.0)
    pre = (agg_ref[0] + agg_ref[1] + x2_ref[...]) * dinv
    o_ref[...] = (jnp.dot(pre, w_ref[...], preferred_element_type=jnp.float32)
                  + b_ref[...])


@jax.jit
def kernel(x, edge_index, W, b):
    n, d = x.shape
    e = edge_index.shape[1]
    assert e % (NW * CH) == 0
    nch = e // (NW * CH)
    npad = ((n + NS * L - 1) // (NS * L)) * (NS * L)

    src3 = edge_index[0].reshape(NW, nch, CH)
    dst3 = edge_index[1].reshape(NW, nch, CH)
    z1 = jnp.zeros((npad,), jnp.float32)
    z2 = jnp.zeros((npad, d), jnp.float32)

    mesh = plsc.VectorSubcoreMesh(core_axis_name="c", subcore_axis_name="s")

    deg = pl.kernel(
        functools.partial(_deg_body, npad, nch),
        out_type=jax.ShapeDtypeStruct((NC, npad), jnp.float32),
        mesh=mesh,
        scratch_types=[
            pltpu.VMEM((nch, CH), jnp.int32),
            pltpu.VMEM((CH,), jnp.float32),
            pltpu.VMEM_SHARED((npad,), jnp.float32),
        ],
    )(dst3, z1)
    degt = deg.T

    nblk = npad // R
    x2 = pl.pallas_call(
        _scale_body,
        grid=(nblk,),
        in_specs=[
            pl.BlockSpec((R, d), lambda i: (i, 0)),
            pl.BlockSpec((R, NC), lambda i: (i, 0)),
        ],
        out_specs=pl.BlockSpec((R, d), lambda i: (i, 0)),
        out_shape=jax.ShapeDtypeStruct((n, d), jnp.float32),
    )(x, degt)

    ei3 = jnp.stack([src3, dst3], axis=2)

    agg = pl.kernel(
        functools.partial(_scatter_body, npad, nch, d),
        out_type=jax.ShapeDtypeStruct((NC, npad, d), jnp.float32),
        mesh=mesh,
        scratch_types=[
            pltpu.VMEM((4, 2, CH), jnp.int32),
            pltpu.VMEM((2, CH, d), jnp.float32),
            pltpu.SemaphoreType.DMA,
            pltpu.SemaphoreType.DMA,
            pltpu.SemaphoreType.DMA,
            pltpu.VMEM_SHARED((npad, d), jnp.float32),
        ],
    )(x2, ei3, z2)

    out = pl.pallas_call(
        _final_body,
        grid=(nblk,),
        in_specs=[
            pl.BlockSpec((NC, R, d), lambda i: (0, i, 0)),
            pl.BlockSpec((R, d), lambda i: (i, 0)),
            pl.BlockSpec((R, NC), lambda i: (i, 0)),
            pl.BlockSpec((d, d), lambda i: (0, 0)),
            pl.BlockSpec((1, d), lambda i: (0, 0)),
        ],
        out_specs=pl.BlockSpec((R, d), lambda i: (i, 0)),
        out_shape=jax.ShapeDtypeStruct((n, d), jnp.float32),
    )(agg, x2, degt, W, b.reshape(1, d))

    return out
